# baseline (speedup 1.0000x reference)
import functools

import jax
import jax.numpy as jnp
from jax import lax
from jax.experimental import pallas as pl
from jax.experimental.pallas import tpu as pltpu
from jax.experimental.pallas import tpu_sc as plsc

_B = 16384
_D = 128
_V = 1000

_info = plsc.get_sparse_core_info()
_NC, _NS = _info.num_cores, _info.num_subcores
_NW = _NC * _NS
_BPW = _B // _NW
_NCHUNK = 8
_C = _BPW // _NCHUNK

_VPT = 64
_VREM = _V - _VPT * (_NS - 1)


@functools.partial(
    pl.kernel,
    mesh=plsc.VectorSubcoreMesh(core_axis_name="c", subcore_axis_name="s"),
    out_type=jax.ShapeDtypeStruct((_B, _D), jnp.float32),
    scratch_types=[
        pltpu.VMEM((_BPW,), jnp.int32),
        pltpu.VMEM((_NCHUNK, _C, _D), jnp.float32),
        pltpu.VMEM_SHARED((_V, _D), jnp.float32),
        pltpu.SemaphoreType.DMA,
        pltpu.SemaphoreType.DMA,
        pltpu.SemaphoreType.DMA,
    ],
)
def _sc_gather(idx_hbm, table_hbm, out_hbm, idx_v, rows_v, tbl_s, gsem, ssem, tsem):
    cid = lax.axis_index("c")
    sid = lax.axis_index("s")
    wid = sid * _NC + cid
    base = wid * _BPW
    icopy = pltpu.async_copy(idx_hbm.at[pl.ds(base, _BPW)], idx_v, tsem)

    half = _VPT // 2

    @pl.when(sid < _NS - 1)
    def _():
        c0 = pltpu.async_copy(
            table_hbm.at[pl.ds(sid * _VPT, half)],
            tbl_s.at[pl.ds(sid * _VPT, half)],
            tsem,
        )
        c1 = pltpu.async_copy(
            table_hbm.at[pl.ds(sid * _VPT + half, half)],
            tbl_s.at[pl.ds(sid * _VPT + half, half)],
            tsem,
        )
        c0.wait()
        c1.wait()

    @pl.when(sid == _NS - 1)
    def _():
        pltpu.sync_copy(
            table_hbm.at[pl.ds(_VPT * (_NS - 1), _VREM)],
            tbl_s.at[pl.ds(_VPT * (_NS - 1), _VREM)],
        )

    icopy.wait()
    plsc.subcore_barrier()
    gathers = []
    for k in range(_NCHUNK):
        gathers.append(
            pltpu.async_copy(
                tbl_s.at[idx_v.at[pl.ds(k * _C, _C)]], rows_v.at[k], gsem
            )
        )
    stores = []
    for k in range(_NCHUNK):
        gathers[k].wait()
        stores.append(
            pltpu.async_copy(
                rows_v.at[k], out_hbm.at[pl.ds(base + k * _C, _C)], ssem
            )
        )
    for k in range(_NCHUNK):
        stores[k].wait()


@jax.jit
def kernel(t, W):
    return _sc_gather(t, W)


# speedup vs baseline: 1.0022x; 1.0022x over previous
import functools

import jax
import jax.numpy as jnp
from jax import lax
from jax.experimental import pallas as pl
from jax.experimental.pallas import tpu as pltpu
from jax.experimental.pallas import tpu_sc as plsc

_B = 16384
_D = 128
_V = 1000

_info = plsc.get_sparse_core_info()
_NC, _NS = _info.num_cores, _info.num_subcores
_NW = _NC * _NS
_BPW = _B // _NW
_NCHUNK = 8
_C = _BPW // _NCHUNK

_VPT = 64
_VREM = _V - _VPT * (_NS - 1)


@functools.partial(
    pl.kernel,
    mesh=plsc.VectorSubcoreMesh(core_axis_name="c", subcore_axis_name="s"),
    out_type=jax.ShapeDtypeStruct((_B, _D), jnp.float32),
    scratch_types=[
        pltpu.VMEM((_BPW,), jnp.int32),
        pltpu.VMEM((_NCHUNK, _C, _D), jnp.float32),
        pltpu.VMEM_SHARED((_V, _D), jnp.float32),
        pltpu.SemaphoreType.DMA,
        pltpu.SemaphoreType.DMA,
        pltpu.SemaphoreType.DMA,
    ],
)
def _sc_gather(idx_hbm, table_hbm, out_hbm, idx_v, rows_v, tbl_s, gsem, ssem, tsem):
    cid = lax.axis_index("c")
    sid = lax.axis_index("s")
    wid = sid * _NC + cid
    base = wid * _BPW
    icopy = pltpu.async_copy(idx_hbm.at[pl.ds(base, _BPW)], idx_v, tsem)

    @pl.when(sid < _NS - 1)
    def _():
        pltpu.sync_copy(
            table_hbm.at[pl.ds(sid * _VPT, _VPT)],
            tbl_s.at[pl.ds(sid * _VPT, _VPT)],
        )

    @pl.when(sid == _NS - 1)
    def _():
        pltpu.sync_copy(
            table_hbm.at[pl.ds(_VPT * (_NS - 1), _VREM)],
            tbl_s.at[pl.ds(_VPT * (_NS - 1), _VREM)],
        )

    icopy.wait()
    plsc.subcore_barrier()
    gathers = []
    for k in range(_NCHUNK):
        gathers.append(
            pltpu.async_copy(
                tbl_s.at[idx_v.at[pl.ds(k * _C, _C)]], rows_v.at[k], gsem
            )
        )
    stores = []
    for k in range(_NCHUNK):
        gathers[k].wait()
        stores.append(
            pltpu.async_copy(
                rows_v.at[k], out_hbm.at[pl.ds(base + k * _C, _C)], ssem
            )
        )
    for k in range(_NCHUNK):
        stores[k].wait()


@jax.jit
def kernel(t, W):
    return _sc_gather(t, W)


# speedup vs baseline: 1.0064x; 1.0042x over previous
import functools

import jax
import jax.numpy as jnp
from jax import lax
from jax.experimental import pallas as pl
from jax.experimental.pallas import tpu as pltpu
from jax.experimental.pallas import tpu_sc as plsc

_B = 16384
_D = 128
_V = 1000

_info = plsc.get_sparse_core_info()
_NC, _NS = _info.num_cores, _info.num_subcores
_NW = _NC * _NS
_BPW = _B // _NW
_NCHUNK = 8
_C = _BPW // _NCHUNK

_VPT = 64
_VREM = _V - _VPT * (_NS - 1)


@functools.partial(
    pl.kernel,
    mesh=plsc.VectorSubcoreMesh(core_axis_name="c", subcore_axis_name="s"),
    out_type=jax.ShapeDtypeStruct((_B, _D), jnp.float32),
    scratch_types=[
        pltpu.VMEM((_BPW,), jnp.int32),
        pltpu.VMEM((_NCHUNK, _C, _D), jnp.float32),
        pltpu.VMEM_SHARED((_V, _D), jnp.float32),
    ]
    + [pltpu.SemaphoreType.DMA] * _NCHUNK
    + [
        pltpu.SemaphoreType.DMA,
        pltpu.SemaphoreType.DMA,
    ],
)
def _sc_gather(idx_hbm, table_hbm, out_hbm, idx_v, rows_v, tbl_s, *sems):
    gsems = sems[:_NCHUNK]
    ssem, tsem = sems[_NCHUNK], sems[_NCHUNK + 1]
    cid = lax.axis_index("c")
    sid = lax.axis_index("s")
    wid = sid * _NC + cid
    base = wid * _BPW
    icopy = pltpu.async_copy(idx_hbm.at[pl.ds(base, _BPW)], idx_v, tsem)

    @pl.when(sid < _NS - 1)
    def _():
        pltpu.sync_copy(
            table_hbm.at[pl.ds(sid * _VPT, _VPT)],
            tbl_s.at[pl.ds(sid * _VPT, _VPT)],
        )

    @pl.when(sid == _NS - 1)
    def _():
        pltpu.sync_copy(
            table_hbm.at[pl.ds(_VPT * (_NS - 1), _VREM)],
            tbl_s.at[pl.ds(_VPT * (_NS - 1), _VREM)],
        )

    icopy.wait()
    plsc.subcore_barrier()
    gathers = []
    for k in range(_NCHUNK):
        gathers.append(
            pltpu.async_copy(
                tbl_s.at[idx_v.at[pl.ds(k * _C, _C)]], rows_v.at[k], gsems[k]
            )
        )
    stores = []
    for k in range(_NCHUNK):
        gathers[k].wait()
        stores.append(
            pltpu.async_copy(
                rows_v.at[k], out_hbm.at[pl.ds(base + k * _C, _C)], ssem
            )
        )
    for k in range(_NCHUNK):
        stores[k].wait()


@jax.jit
def kernel(t, W):
    return _sc_gather(t, W)
